# baseline (speedup 1.0000x reference)
import functools

import jax
import jax.numpy as jnp
from jax import lax
from jax.experimental import pallas as pl
from jax.experimental.pallas import tpu as pltpu
from jax.experimental.pallas import tpu_sc as plsc

EMB = 64
NC = 2
NS = 16
NW = NC * NS
BLK = 128
NBUF = 4

_mesh = plsc.VectorSubcoreMesh(core_axis_name="c", subcore_axis_name="s")


def _make_gather(seq: int):
    et = EMB // 8

    @functools.partial(
        pl.kernel,
        mesh=_mesh,
        out_type=jax.ShapeDtypeStruct((seq, et, NW, 8, BLK), jnp.float32),
        scratch_types=[
            pltpu.VMEM((seq, BLK), jnp.int32),
            pltpu.VMEM((NBUF, BLK, EMB), jnp.float32),
            pltpu.VMEM((NBUF, et, 8, BLK), jnp.float32),
            pltpu.SemaphoreType.DMA((NBUF,)),
            pltpu.SemaphoreType.DMA((NBUF,)),
        ],
        compiler_params=pltpu.CompilerParams(
            use_tc_tiling_on_sc=False, needs_layout_passes=False
        ),
    )
    def gather_kernel(ids_hbm, table_hbm, out_hbm, idx_v, rbufs, tbufs, gsems, osems):
        wid = lax.axis_index("s") * NC + lax.axis_index("c")

        pltpu.sync_copy(ids_hbm.at[:, pl.ds(wid * BLK, BLK)], idx_v)

        rowidx = [lax.iota(jnp.int32, 16) + blk * 16 for blk in range(8)]

        for b in range(NBUF):
            pltpu.async_copy(table_hbm.at[idx_v.at[b]], rbufs.at[b], gsems.at[b])

        def step(i, _):
            b = lax.rem(i, NBUF)

            @pl.when(i >= NBUF)
            def _wait_out():
                pltpu.make_async_copy(
                    tbufs.at[b], out_hbm.at[0, :, wid], osems.at[b]
                ).wait()

            pltpu.make_async_copy(
                table_hbm.at[idx_v.at[i]], rbufs.at[b], gsems.at[b]
            ).wait()

            bvec = jnp.full((16,), b, jnp.int32)
            for c in range(EMB):
                colvec = jnp.full((16,), c, jnp.int32)
                for blk in range(8):
                    v = plsc.load_gather(rbufs, [bvec, rowidx[blk], colvec])
                    tbufs[b, c // 8, c % 8, pl.ds(blk * 16, 16)] = v

            pltpu.async_copy(tbufs.at[b], out_hbm.at[i, :, wid], osems.at[b])

            @pl.when(i + NBUF < seq)
            def _refill():
                pltpu.async_copy(
                    table_hbm.at[idx_v.at[i + NBUF]], rbufs.at[b], gsems.at[b]
                )

            return _

        lax.fori_loop(0, seq, step, None)

        for b in range(NBUF):
            pltpu.make_async_copy(
                tbufs.at[b], out_hbm.at[0, :, wid], osems.at[b]
            ).wait()

    return gather_kernel


def kernel(input_ids, token_table):
    batch, seq = input_ids.shape
    ids_t = jnp.transpose(input_ids.astype(jnp.int32))
    out5 = _make_gather(seq)(ids_t, token_table)
    return jnp.transpose(out5, (2, 4, 0, 1, 3)).reshape(batch, seq, EMB)


# speedup vs baseline: 1.4042x; 1.4042x over previous
import functools

import jax
import jax.numpy as jnp
from jax import lax
from jax.experimental import pallas as pl
from jax.experimental.pallas import tpu as pltpu
from jax.experimental.pallas import tpu_sc as plsc

EMB = 64
NC = 2
NS = 16
NW = NC * NS
BLK = 128
NBUF = 4

_mesh = plsc.VectorSubcoreMesh(core_axis_name="c", subcore_axis_name="s")


def _make_gather(seq: int):
    et = EMB // 8

    @functools.partial(
        pl.kernel,
        mesh=_mesh,
        out_type=jax.ShapeDtypeStruct((seq, et, NW, 8, BLK), jnp.float32),
        scratch_types=[
            pltpu.VMEM((seq, BLK), jnp.int32),
            pltpu.VMEM((NBUF, BLK, EMB), jnp.float32),
            pltpu.VMEM((NBUF, et, 8, BLK), jnp.float32),
            pltpu.SemaphoreType.DMA((NBUF,)),
            pltpu.SemaphoreType.DMA((NBUF,)),
        ],
        compiler_params=pltpu.CompilerParams(
            use_tc_tiling_on_sc=False, needs_layout_passes=False
        ),
    )
    def gather_kernel(ids_hbm, table_hbm, out_hbm, idx_v, rbufs, tbufs, gsems, osems):
        wid = lax.axis_index("s") * NC + lax.axis_index("c")

        pltpu.sync_copy(ids_hbm.at[:, pl.ds(wid * BLK, BLK)], idx_v)

        rowidx = [lax.iota(jnp.int32, 16) + blk * 16 for blk in range(8)]

        for b in range(NBUF):
            pltpu.async_copy(table_hbm.at[idx_v.at[b]], rbufs.at[b], gsems.at[b])

        def step(i, _):
            b = lax.rem(i, NBUF)

            @pl.when(i >= NBUF)
            def _wait_out():
                pltpu.make_async_copy(
                    tbufs.at[b], out_hbm.at[0, :, wid], osems.at[b]
                ).wait()

            pltpu.make_async_copy(
                table_hbm.at[idx_v.at[i]], rbufs.at[b], gsems.at[b]
            ).wait()

            bvec = jnp.full((16,), b, jnp.int32)

            @plsc.parallel_loop(0, EMB, unroll=8)
            def _transpose(c):
                colvec = jnp.full((16,), c, jnp.int32)
                for blk in range(8):
                    v = plsc.load_gather(rbufs, [bvec, rowidx[blk], colvec])
                    tbufs[b, c // 8, c % 8, pl.ds(blk * 16, 16)] = v

            pltpu.async_copy(tbufs.at[b], out_hbm.at[i, :, wid], osems.at[b])

            @pl.when(i + NBUF < seq)
            def _refill():
                pltpu.async_copy(
                    table_hbm.at[idx_v.at[i + NBUF]], rbufs.at[b], gsems.at[b]
                )

            return _

        lax.fori_loop(0, seq, step, None)

        for b in range(NBUF):
            pltpu.make_async_copy(
                tbufs.at[b], out_hbm.at[0, :, wid], osems.at[b]
            ).wait()

    return gather_kernel


def kernel(input_ids, token_table):
    batch, seq = input_ids.shape
    ids_t = jnp.transpose(input_ids.astype(jnp.int32))
    out5 = _make_gather(seq)(ids_t, token_table)
    return jnp.transpose(out5, (2, 4, 0, 1, 3)).reshape(batch, seq, EMB)


# speedup vs baseline: 2.4492x; 1.7443x over previous
import functools

import jax
import jax.numpy as jnp
from jax import lax
from jax.experimental import pallas as pl
from jax.experimental.pallas import tpu as pltpu
from jax.experimental.pallas import tpu_sc as plsc

EMB = 64
NC = 2
NS = 16
NW = NC * NS
BLK = 128
NBUF = 4

_mesh = plsc.VectorSubcoreMesh(core_axis_name="c", subcore_axis_name="s")


def _make_gather(seq: int):
    et = EMB // 8

    @functools.partial(
        pl.kernel,
        mesh=_mesh,
        out_type=jax.ShapeDtypeStruct((seq, et, NW, 8, BLK), jnp.float32),
        scratch_types=[
            pltpu.VMEM((seq, BLK), jnp.int32),
            pltpu.VMEM((NBUF, BLK, EMB), jnp.float32),
            pltpu.VMEM((NBUF, et, 8, BLK + 1), jnp.float32),
            pltpu.SemaphoreType.DMA((NBUF,)),
            pltpu.SemaphoreType.DMA((NBUF,)),
        ],
        compiler_params=pltpu.CompilerParams(
            use_tc_tiling_on_sc=False, needs_layout_passes=False
        ),
    )
    def gather_kernel(ids_hbm, table_hbm, out_hbm, idx_v, rbufs, tbufs, gsems, osems):
        wid = lax.axis_index("s") * NC + lax.axis_index("c")

        pltpu.sync_copy(ids_hbm.at[:, pl.ds(wid * BLK, BLK)], idx_v)

        lanes = lax.iota(jnp.int32, 16)
        e8vec = [(lanes + q * 16) // 8 for q in range(EMB // 16)]
        elvec = [(lanes + q * 16) % 8 for q in range(EMB // 16)]

        for b in range(NBUF):
            pltpu.async_copy(table_hbm.at[idx_v.at[b]], rbufs.at[b], gsems.at[b])

        def step(i, _):
            b = lax.rem(i, NBUF)

            @pl.when(i >= NBUF)
            def _wait_out():
                pltpu.make_async_copy(
                    tbufs.at[b, :, :, pl.ds(0, BLK)],
                    out_hbm.at[0, :, wid],
                    osems.at[b],
                ).wait()

            pltpu.make_async_copy(
                table_hbm.at[idx_v.at[i]], rbufs.at[b], gsems.at[b]
            ).wait()

            bvec = jnp.full((16,), b, jnp.int32)

            @plsc.parallel_loop(0, BLK, unroll=8)
            def _transpose(r):
                rvec = jnp.full((16,), r, jnp.int32)
                for q in range(EMB // 16):
                    v = rbufs[b, r, pl.ds(q * 16, 16)]
                    plsc.store_scatter(tbufs, [bvec, e8vec[q], elvec[q], rvec], v)

            pltpu.async_copy(
                tbufs.at[b, :, :, pl.ds(0, BLK)], out_hbm.at[i, :, wid], osems.at[b]
            )

            @pl.when(i + NBUF < seq)
            def _refill():
                pltpu.async_copy(
                    table_hbm.at[idx_v.at[i + NBUF]], rbufs.at[b], gsems.at[b]
                )

            return _

        lax.fori_loop(0, seq, step, None)

        for b in range(NBUF):
            pltpu.make_async_copy(
                tbufs.at[b, :, :, pl.ds(0, BLK)],
                out_hbm.at[0, :, wid],
                osems.at[b],
            ).wait()

    return gather_kernel


def kernel(input_ids, token_table):
    batch, seq = input_ids.shape
    ids_t = jnp.transpose(input_ids.astype(jnp.int32))
    out5 = _make_gather(seq)(ids_t, token_table)
    return jnp.transpose(out5, (2, 4, 0, 1, 3)).reshape(batch, seq, EMB)
